# baseline (speedup 1.0000x reference)
import functools

import jax
import jax.numpy as jnp
from jax import lax
from jax.experimental import pallas as pl
from jax.experimental.pallas import tpu as pltpu
from jax.experimental.pallas import tpu_sc as plsc

NC = 2
NS = 16
NW = NC * NS
LANES = 16

CHUNK_ROWS = 2
UNROLL = 4


def _sc_partials(pred, gt, rows, hw, nj):
    rows_per_w = rows // NW
    chunks = rows_per_w // CHUNK_ROWS
    cw = CHUNK_ROWS * hw
    vecs = hw // LANES
    acc_w = nj * LANES

    predc = pred.reshape(rows // CHUNK_ROWS, cw)
    gtc = gt.reshape(rows // CHUNK_ROWS, cw)

    mesh = plsc.VectorSubcoreMesh(core_axis_name="c", subcore_axis_name="s")

    @functools.partial(
        pl.kernel,
        out_type=jax.ShapeDtypeStruct((NW, 4, acc_w), jnp.float32),
        mesh=mesh,
        scratch_types=[
            pltpu.VMEM((2, cw), jnp.float32),
            pltpu.VMEM((2, cw), jnp.float32),
            pltpu.VMEM((4, acc_w), jnp.float32),
            pltpu.SemaphoreType.DMA((2,)),
            pltpu.SemaphoreType.DMA((2,)),
        ],
    )
    def body(pred_hbm, gt_hbm, out_hbm, pbuf, gbuf, acc, psem, gsem):
        wid = lax.axis_index("s") * NC + lax.axis_index("c")
        cbase = wid * chunks

        def start(c_local, slot):
            gc = cbase + c_local
            pltpu.make_async_copy(pred_hbm.at[gc], pbuf.at[slot], psem.at[slot]).start()
            pltpu.make_async_copy(gt_hbm.at[gc], gbuf.at[slot], gsem.at[slot]).start()

        def wait(c_local, slot):
            gc = cbase + c_local
            pltpu.make_async_copy(pred_hbm.at[gc], pbuf.at[slot], psem.at[slot]).wait()
            pltpu.make_async_copy(gt_hbm.at[gc], gbuf.at[slot], gsem.at[slot]).wait()

        zero16 = jnp.zeros((LANES,), jnp.float32)
        ninf16 = jnp.full((LANES,), -jnp.inf, jnp.float32)
        pinf16 = jnp.full((LANES,), jnp.inf, jnp.float32)
        for j in range(nj):
            sl = pl.ds(j * LANES, LANES)
            acc[0, sl] = zero16
            acc[1, sl] = zero16
            acc[2, sl] = ninf16
            acc[3, sl] = pinf16

        start(0, 0)
        start(1, 1)

        def do_chunk(c_local, slot):
            wait(c_local, slot)
            row0 = c_local * CHUNK_ROWS
            for rr in range(CHUNK_ROWS):
                j = (row0 + rr) % nj
                base_off = rr * hw

                def inner(i, carry, _base=base_off, _slot=slot):
                    ps, pc, nm, ma = carry
                    for u in range(UNROLL):
                        off = _base + (i * UNROLL + u) * LANES
                        p = pbuf[_slot, pl.ds(off, LANES)]
                        g = gbuf[_slot, pl.ds(off, LANES)]
                        d = p - g
                        sq = d * d
                        posm = g > 0.0
                        ps = ps + jnp.where(posm, sq, 0.0)
                        pc = pc + jnp.where(posm, 1.0, 0.0)
                        negm = g == 0.0
                        nm = jnp.maximum(nm, jnp.where(negm, p, -jnp.inf))
                        ma = jnp.minimum(ma, jnp.abs(g))
                    return ps, pc, nm, ma

                ps, pc, nm, ma = lax.fori_loop(
                    0, vecs // UNROLL, inner, (zero16, zero16, ninf16, pinf16))
                sl = pl.ds(j * LANES, LANES)
                acc[0, sl] = acc[0, sl] + ps
                acc[1, sl] = acc[1, sl] + pc
                acc[2, sl] = jnp.maximum(acc[2, sl], nm)
                acc[3, sl] = jnp.minimum(acc[3, sl], ma)

            @pl.when(c_local + 2 < chunks)
            def _():
                start(c_local + 2, slot)

        def outer(c2, carry):
            do_chunk(c2 * 2, 0)
            do_chunk(c2 * 2 + 1, 1)
            return carry

        lax.fori_loop(0, chunks // 2, outer, 0)

        pltpu.sync_copy(acc, out_hbm.at[wid])

    return body(predc, gtc)


def _finalize(parts, nj):
    def fin_body(x_ref, o_ref):
        x = x_ref[...]
        ps = jnp.sum(x[0 * nj:1 * nj, :], axis=1, keepdims=True)
        pc = jnp.sum(x[1 * nj:2 * nj, :], axis=1, keepdims=True)
        nm = jnp.max(x[2 * nj:3 * nj, :], axis=1, keepdims=True)
        ma = jnp.min(x[3 * nj:4 * nj, :], axis=1, keepdims=True)
        present = ma == 0.0
        nm_safe = jnp.where(present, nm, 0.0)
        loss_j = ps / jnp.maximum(pc, 1.0) + nm_safe * nm_safe
        o_ref[...] = jnp.sum(loss_j, axis=0, keepdims=True) / nj

    return pl.pallas_call(
        fin_body,
        out_shape=jax.ShapeDtypeStruct((1, 1), jnp.float32),
    )(parts)


@jax.jit
def kernel(output, target):
    b, nj = output.shape[0], output.shape[1]
    hw = output.shape[2] * output.shape[3]
    rows = b * nj
    pred = output.reshape(rows, hw)
    gt = target.reshape(rows, hw)

    parts = _sc_partials(pred, gt, rows, hw, nj)
    parts = parts.reshape(NW, 4, nj, LANES)
    parts = parts.transpose(1, 2, 0, 3).reshape(4 * nj, NW * LANES)
    loss = _finalize(parts, nj)
    return loss[0, 0]


# speedup vs baseline: 1.0531x; 1.0531x over previous
import functools

import jax
import jax.numpy as jnp
from jax import lax
from jax.experimental import pallas as pl
from jax.experimental.pallas import tpu as pltpu
from jax.experimental.pallas import tpu_sc as plsc

NC = 2
NS = 16
NW = NC * NS
LANES = 16

CHUNK_ROWS = 2
UNROLL = 4


def _sc_partials(pred, gt, rows, hw, nj):
    rows_per_w = rows // NW
    chunks = rows_per_w // CHUNK_ROWS
    cw = CHUNK_ROWS * hw
    vecs = hw // LANES
    acc_w = nj * LANES

    predc = pred.reshape(rows * hw)
    gtc = gt.reshape(rows * hw)

    mesh = plsc.VectorSubcoreMesh(core_axis_name="c", subcore_axis_name="s")

    @functools.partial(
        pl.kernel,
        out_type=jax.ShapeDtypeStruct((NW, 4 * acc_w), jnp.float32),
        mesh=mesh,
        scratch_types=[
            pltpu.VMEM((cw,), jnp.float32),
            pltpu.VMEM((cw,), jnp.float32),
            pltpu.VMEM((cw,), jnp.float32),
            pltpu.VMEM((cw,), jnp.float32),
            pltpu.VMEM((4 * acc_w,), jnp.float32),
            pltpu.SemaphoreType.DMA,
            pltpu.SemaphoreType.DMA,
            pltpu.SemaphoreType.DMA,
            pltpu.SemaphoreType.DMA,
        ],
    )
    def body(pred_hbm, gt_hbm, out_hbm, pbuf0, pbuf1, gbuf0, gbuf1, acc,
             psem0, psem1, gsem0, gsem1):
        wid = lax.axis_index("s") * NC + lax.axis_index("c")
        cbase = wid * chunks
        pbufs = (pbuf0, pbuf1)
        gbufs = (gbuf0, gbuf1)
        psems = (psem0, psem1)
        gsems = (gsem0, gsem1)

        def start(c_local, slot):
            off = (cbase + c_local) * cw
            pltpu.make_async_copy(pred_hbm.at[pl.ds(off, cw)], pbufs[slot],
                                  psems[slot]).start()
            pltpu.make_async_copy(gt_hbm.at[pl.ds(off, cw)], gbufs[slot],
                                  gsems[slot]).start()

        def wait(c_local, slot):
            off = (cbase + c_local) * cw
            pltpu.make_async_copy(pred_hbm.at[pl.ds(off, cw)], pbufs[slot],
                                  psems[slot]).wait()
            pltpu.make_async_copy(gt_hbm.at[pl.ds(off, cw)], gbufs[slot],
                                  gsems[slot]).wait()

        zero16 = jnp.zeros((LANES,), jnp.float32)
        ninf16 = jnp.full((LANES,), -jnp.inf, jnp.float32)
        pinf16 = jnp.full((LANES,), jnp.inf, jnp.float32)
        for j in range(nj):
            acc[pl.ds(0 * acc_w + j * LANES, LANES)] = zero16
            acc[pl.ds(1 * acc_w + j * LANES, LANES)] = zero16
            acc[pl.ds(2 * acc_w + j * LANES, LANES)] = ninf16
            acc[pl.ds(3 * acc_w + j * LANES, LANES)] = pinf16

        start(0, 0)
        start(1, 1)

        def do_chunk(c_local, slot):
            wait(c_local, slot)
            row0 = c_local * CHUNK_ROWS
            for rr in range(CHUNK_ROWS):
                j = (row0 + rr) % nj
                base_off = rr * hw

                def inner(i, carry, _base=base_off, _slot=slot):
                    ps, pc, nm, ma = carry
                    for u in range(UNROLL):
                        off = _base + (i * UNROLL + u) * LANES
                        p = pbufs[_slot][pl.ds(off, LANES)]
                        g = gbufs[_slot][pl.ds(off, LANES)]
                        d = p - g
                        sq = d * d
                        posm = g > 0.0
                        ps = ps + jnp.where(posm, sq, 0.0)
                        pc = pc + jnp.where(posm, 1.0, 0.0)
                        negm = g == 0.0
                        nm = jnp.maximum(nm, jnp.where(negm, p, -jnp.inf))
                        ma = jnp.minimum(ma, jnp.abs(g))
                    return ps, pc, nm, ma

                ps, pc, nm, ma = lax.fori_loop(
                    0, vecs // UNROLL, inner, (zero16, zero16, ninf16, pinf16))
                s0 = pl.ds(0 * acc_w + j * LANES, LANES)
                s1 = pl.ds(1 * acc_w + j * LANES, LANES)
                s2 = pl.ds(2 * acc_w + j * LANES, LANES)
                s3 = pl.ds(3 * acc_w + j * LANES, LANES)
                acc[s0] = acc[s0] + ps
                acc[s1] = acc[s1] + pc
                acc[s2] = jnp.maximum(acc[s2], nm)
                acc[s3] = jnp.minimum(acc[s3], ma)

            @pl.when(c_local + 2 < chunks)
            def _():
                start(c_local + 2, slot)

        def outer(c2, carry):
            do_chunk(c2 * 2, 0)
            do_chunk(c2 * 2 + 1, 1)
            return carry

        lax.fori_loop(0, chunks // 2, outer, 0)

        pltpu.sync_copy(acc, out_hbm.at[wid])

    return body(predc, gtc)


def _finalize(parts, nj):
    def fin_body(x_ref, o_ref):
        x = x_ref[...]
        ps = jnp.sum(x[0 * nj:1 * nj, :], axis=1, keepdims=True)
        pc = jnp.sum(x[1 * nj:2 * nj, :], axis=1, keepdims=True)
        nm = jnp.max(x[2 * nj:3 * nj, :], axis=1, keepdims=True)
        ma = jnp.min(x[3 * nj:4 * nj, :], axis=1, keepdims=True)
        present = ma == 0.0
        nm_safe = jnp.where(present, nm, 0.0)
        loss_j = ps / jnp.maximum(pc, 1.0) + nm_safe * nm_safe
        o_ref[...] = jnp.sum(loss_j, axis=0, keepdims=True) / nj

    return pl.pallas_call(
        fin_body,
        out_shape=jax.ShapeDtypeStruct((1, 1), jnp.float32),
    )(parts)


@jax.jit
def kernel(output, target):
    b, nj = output.shape[0], output.shape[1]
    hw = output.shape[2] * output.shape[3]
    rows = b * nj
    pred = output.reshape(rows, hw)
    gt = target.reshape(rows, hw)

    parts = _sc_partials(pred, gt, rows, hw, nj)
    parts = parts.reshape(NW, 4, nj, LANES)
    parts = parts.transpose(1, 2, 0, 3).reshape(4 * nj, NW * LANES)
    loss = _finalize(parts, nj)
    return loss[0, 0]


# speedup vs baseline: 1.5629x; 1.4841x over previous
import functools

import jax
import jax.numpy as jnp
from jax import lax
from jax.experimental import pallas as pl
from jax.experimental.pallas import tpu as pltpu
from jax.experimental.pallas import tpu_sc as plsc

NC = 2
NS = 16
NW = NC * NS
LANES = 16

CHUNK_ROWS = 2
UNROLL = 4


def _sc_partials(pred, gt, b, nj, h, w):
    bpw = b // NW
    imgs = bpw * nj
    vecs_r = w // LANES
    acc_w = nj * LANES

    mesh = plsc.VectorSubcoreMesh(core_axis_name="c", subcore_axis_name="s")

    @functools.partial(
        pl.kernel,
        out_type=jax.ShapeDtypeStruct((NW, 4 * acc_w), jnp.float32),
        mesh=mesh,
        scratch_types=[
            pltpu.VMEM((h, w), jnp.float32),
            pltpu.VMEM((h, w), jnp.float32),
            pltpu.VMEM((h, w), jnp.float32),
            pltpu.VMEM((h, w), jnp.float32),
            pltpu.VMEM((4 * acc_w,), jnp.float32),
            pltpu.SemaphoreType.DMA,
            pltpu.SemaphoreType.DMA,
            pltpu.SemaphoreType.DMA,
            pltpu.SemaphoreType.DMA,
        ],
    )
    def body(pred_hbm, gt_hbm, out_hbm, pbuf0, pbuf1, gbuf0, gbuf1, acc,
             psem0, psem1, gsem0, gsem1):
        wid = lax.axis_index("s") * NC + lax.axis_index("c")
        b0 = wid * bpw
        pbufs = (pbuf0, pbuf1)
        gbufs = (gbuf0, gbuf1)
        psems = (psem0, psem1)
        gsems = (gsem0, gsem1)

        def start(i_local, slot):
            bi = b0 + i_local // nj
            ji = i_local % nj
            pltpu.make_async_copy(pred_hbm.at[bi, ji], pbufs[slot],
                                  psems[slot]).start()
            pltpu.make_async_copy(gt_hbm.at[bi, ji], gbufs[slot],
                                  gsems[slot]).start()

        def wait(i_local, slot):
            bi = b0 + i_local // nj
            ji = i_local % nj
            pltpu.make_async_copy(pred_hbm.at[bi, ji], pbufs[slot],
                                  psems[slot]).wait()
            pltpu.make_async_copy(gt_hbm.at[bi, ji], gbufs[slot],
                                  gsems[slot]).wait()

        zero16 = jnp.zeros((LANES,), jnp.float32)
        ninf16 = jnp.full((LANES,), -jnp.inf, jnp.float32)
        pinf16 = jnp.full((LANES,), jnp.inf, jnp.float32)
        for j in range(nj):
            acc[pl.ds(0 * acc_w + j * LANES, LANES)] = zero16
            acc[pl.ds(1 * acc_w + j * LANES, LANES)] = zero16
            acc[pl.ds(2 * acc_w + j * LANES, LANES)] = ninf16
            acc[pl.ds(3 * acc_w + j * LANES, LANES)] = pinf16

        start(0, 0)
        start(1, 1)

        def do_img(i_local, slot):
            wait(i_local, slot)
            j = i_local % nj

            def inner(r, carry, _slot=slot):
                ps, pc, nm, ma = carry
                for u in range(vecs_r):
                    p = pbufs[_slot][r, pl.ds(u * LANES, LANES)]
                    g = gbufs[_slot][r, pl.ds(u * LANES, LANES)]
                    d = p - g
                    sq = d * d
                    posm = g > 0.0
                    ps = ps + jnp.where(posm, sq, 0.0)
                    pc = pc + jnp.where(posm, 1.0, 0.0)
                    negm = g == 0.0
                    nm = jnp.maximum(nm, jnp.where(negm, p, -jnp.inf))
                    ma = jnp.minimum(ma, jnp.abs(g))
                return ps, pc, nm, ma

            ps, pc, nm, ma = lax.fori_loop(
                0, h, inner, (zero16, zero16, ninf16, pinf16))
            s0 = pl.ds(0 * acc_w + j * LANES, LANES)
            s1 = pl.ds(1 * acc_w + j * LANES, LANES)
            s2 = pl.ds(2 * acc_w + j * LANES, LANES)
            s3 = pl.ds(3 * acc_w + j * LANES, LANES)
            acc[s0] = acc[s0] + ps
            acc[s1] = acc[s1] + pc
            acc[s2] = jnp.maximum(acc[s2], nm)
            acc[s3] = jnp.minimum(acc[s3], ma)

            @pl.when(i_local + 2 < imgs)
            def _():
                start(i_local + 2, slot)

        def outer(i2, carry):
            do_img(i2 * 2, 0)
            do_img(i2 * 2 + 1, 1)
            return carry

        lax.fori_loop(0, imgs // 2, outer, 0)

        pltpu.sync_copy(acc, out_hbm.at[wid])

    return body(pred, gt)


def _finalize(parts, nj):
    def fin_body(x_ref, o_ref):
        x = x_ref[...]
        ps = jnp.sum(x[0 * nj:1 * nj, :], axis=1, keepdims=True)
        pc = jnp.sum(x[1 * nj:2 * nj, :], axis=1, keepdims=True)
        nm = jnp.max(x[2 * nj:3 * nj, :], axis=1, keepdims=True)
        ma = jnp.min(x[3 * nj:4 * nj, :], axis=1, keepdims=True)
        present = ma == 0.0
        nm_safe = jnp.where(present, nm, 0.0)
        loss_j = ps / jnp.maximum(pc, 1.0) + nm_safe * nm_safe
        o_ref[...] = jnp.sum(loss_j, axis=0, keepdims=True) / nj

    return pl.pallas_call(
        fin_body,
        out_shape=jax.ShapeDtypeStruct((1, 1), jnp.float32),
    )(parts)


@jax.jit
def kernel(output, target):
    b, nj, h, w = output.shape

    parts = _sc_partials(output, target, b, nj, h, w)
    parts = parts.reshape(NW, 4, nj, LANES)
    parts = parts.transpose(1, 2, 0, 3).reshape(4 * nj, NW * LANES)
    loss = _finalize(parts, nj)
    return loss[0, 0]


# speedup vs baseline: 1.6009x; 1.0243x over previous
import functools

import jax
import jax.numpy as jnp
from jax import lax
from jax.experimental import pallas as pl
from jax.experimental.pallas import tpu as pltpu
from jax.experimental.pallas import tpu_sc as plsc

NC = 2
NS = 16
NW = NC * NS
LANES = 16

NBUF = 4


def _sc_partials(pred, gt, b, nj, h, w):
    bpw = b // NW
    imgs = bpw * nj
    vecs_r = w // LANES
    acc_w = nj * LANES

    mesh = plsc.VectorSubcoreMesh(core_axis_name="c", subcore_axis_name="s")

    @functools.partial(
        pl.kernel,
        out_type=jax.ShapeDtypeStruct((NW, 4 * acc_w), jnp.float32),
        mesh=mesh,
        scratch_types=(
            [pltpu.VMEM((h, w), jnp.float32)] * (2 * NBUF)
            + [pltpu.VMEM((4 * acc_w,), jnp.float32)]
            + [pltpu.SemaphoreType.DMA] * (2 * NBUF)
        ),
    )
    def body(pred_hbm, gt_hbm, out_hbm, *rest):
        pbufs = rest[0:NBUF]
        gbufs = rest[NBUF:2 * NBUF]
        acc = rest[2 * NBUF]
        psems = rest[2 * NBUF + 1:3 * NBUF + 1]
        gsems = rest[3 * NBUF + 1:4 * NBUF + 1]
        wid = lax.axis_index("s") * NC + lax.axis_index("c")
        b0 = wid * bpw

        def start(i_local, slot):
            bi = b0 + i_local // nj
            ji = i_local % nj
            pltpu.make_async_copy(pred_hbm.at[bi, ji], pbufs[slot],
                                  psems[slot]).start()
            pltpu.make_async_copy(gt_hbm.at[bi, ji], gbufs[slot],
                                  gsems[slot]).start()

        def wait(i_local, slot):
            bi = b0 + i_local // nj
            ji = i_local % nj
            pltpu.make_async_copy(pred_hbm.at[bi, ji], pbufs[slot],
                                  psems[slot]).wait()
            pltpu.make_async_copy(gt_hbm.at[bi, ji], gbufs[slot],
                                  gsems[slot]).wait()

        zero16 = jnp.zeros((LANES,), jnp.float32)
        ninf16 = jnp.full((LANES,), -jnp.inf, jnp.float32)
        pinf16 = jnp.full((LANES,), jnp.inf, jnp.float32)
        for j in range(nj):
            acc[pl.ds(0 * acc_w + j * LANES, LANES)] = zero16
            acc[pl.ds(1 * acc_w + j * LANES, LANES)] = zero16
            acc[pl.ds(2 * acc_w + j * LANES, LANES)] = ninf16
            acc[pl.ds(3 * acc_w + j * LANES, LANES)] = pinf16

        for s in range(NBUF):
            start(s, s)

        def do_img(i_local, slot):
            wait(i_local, slot)
            j = i_local % nj

            def inner(r, carry, _slot=slot):
                ps, pc, nm, ma = carry
                for u in range(vecs_r):
                    p = pbufs[_slot][r, pl.ds(u * LANES, LANES)]
                    g = gbufs[_slot][r, pl.ds(u * LANES, LANES)]
                    d = p - g
                    sq = d * d
                    posm = g > 0.0
                    ps = ps + jnp.where(posm, sq, 0.0)
                    pc = pc + jnp.where(posm, 1.0, 0.0)
                    negm = g == 0.0
                    nm = jnp.maximum(nm, jnp.where(negm, p, -jnp.inf))
                    ma = jnp.minimum(ma, jnp.abs(g))
                return ps, pc, nm, ma

            ps, pc, nm, ma = lax.fori_loop(
                0, h, inner, (zero16, zero16, ninf16, pinf16))
            s0 = pl.ds(0 * acc_w + j * LANES, LANES)
            s1 = pl.ds(1 * acc_w + j * LANES, LANES)
            s2 = pl.ds(2 * acc_w + j * LANES, LANES)
            s3 = pl.ds(3 * acc_w + j * LANES, LANES)
            acc[s0] = acc[s0] + ps
            acc[s1] = acc[s1] + pc
            acc[s2] = jnp.maximum(acc[s2], nm)
            acc[s3] = jnp.minimum(acc[s3], ma)

            @pl.when(i_local + NBUF < imgs)
            def _():
                start(i_local + NBUF, slot)

        def outer(i2, carry):
            for s in range(NBUF):
                do_img(i2 * NBUF + s, s)
            return carry

        lax.fori_loop(0, imgs // NBUF, outer, 0)

        pltpu.sync_copy(acc, out_hbm.at[wid])

    return body(pred, gt)


def _finalize(parts, nj):
    def fin_body(x_ref, o_ref):
        x = x_ref[...]
        ps = jnp.sum(x[0 * nj:1 * nj, :], axis=1, keepdims=True)
        pc = jnp.sum(x[1 * nj:2 * nj, :], axis=1, keepdims=True)
        nm = jnp.max(x[2 * nj:3 * nj, :], axis=1, keepdims=True)
        ma = jnp.min(x[3 * nj:4 * nj, :], axis=1, keepdims=True)
        present = ma == 0.0
        nm_safe = jnp.where(present, nm, 0.0)
        loss_j = ps / jnp.maximum(pc, 1.0) + nm_safe * nm_safe
        o_ref[...] = jnp.sum(loss_j, axis=0, keepdims=True) / nj

    return pl.pallas_call(
        fin_body,
        out_shape=jax.ShapeDtypeStruct((1, 1), jnp.float32),
    )(parts)


@jax.jit
def kernel(output, target):
    b, nj, h, w = output.shape

    parts = _sc_partials(output, target, b, nj, h, w)
    parts = parts.reshape(NW, 4, nj, LANES)
    parts = parts.transpose(1, 2, 0, 3).reshape(4 * nj, NW * LANES)
    loss = _finalize(parts, nj)
    return loss[0, 0]
